# Initial kernel scaffold; baseline (speedup 1.0000x reference)
import jax
import jax.numpy as jnp
from jax.experimental import pallas as pl


def kernel(x, edge_index, W_in, b_in, convW, convb, alpha, beta, W_out, b_out):
    raise NotImplementedError("write your pallas kernel here")


# speedup vs baseline: 7.7616x; 7.7616x over previous
import jax
import jax.numpy as jnp
from jax import lax
from jax.experimental import pallas as pl
from jax.experimental.pallas import tpu as pltpu
from jax.experimental.pallas import tpu_sc as plsc

N = 10000
E = 320000
D_IN = 128
H = 128
OUT = 40
L = 4

NC = 2
NS = 16
NW = NC * NS

CH = 128
Np = 10240
Ep = NW * 80 * CH
EPT = Ep // NW
CHUNKS = EPT // CH
RPT = Np // NS
DW = 16
NBUF = 4

f32 = jnp.float32

_mesh = plsc.VectorSubcoreMesh(core_axis_name="c", subcore_axis_name="s")


def _sc_deg_body(dst_hbm, out_hbm, ones_v, zero_v,
                 idx0, idx1, idx2, idx3, acc,
                 isem0, isem1, isem2, isem3):
    c = lax.axis_index("c")
    s = lax.axis_index("s")
    wid = c * NS + s
    idx = (idx0, idx1, idx2, idx3)
    isem = (isem0, isem1, isem2, isem3)

    one_row = jnp.ones((DW,), f32)
    zero_row = jnp.zeros((DW,), f32)

    def fill(i, _):
        ones_v[i] = one_row
        zero_v[i] = zero_row
        return 0

    lax.fori_loop(0, CH, fill, 0)

    def istart(j, b):
        base = pl.multiple_of(wid * EPT + j * CH, 8)
        pltpu.async_copy(dst_hbm.at[pl.ds(base, CH)], idx[b], isem[b])

    def iwait(j, b):
        base = pl.multiple_of(wid * EPT + j * CH, 8)
        pltpu.make_async_copy(dst_hbm.at[pl.ds(base, CH)], idx[b],
                              isem[b]).wait()

    def zloop(k, _):
        r = pl.multiple_of(s * RPT + k * CH, 8)
        pltpu.sync_copy(zero_v, acc.at[pl.ds(r, CH)])
        return 0

    lax.fori_loop(0, RPT // CH, zloop, 0)
    plsc.subcore_barrier()

    for b in range(NBUF - 1):
        istart(b, b)

    def chunk(t, _):
        for u in range(NBUF):
            j = NBUF * t + u
            b = u
            bn = (u + NBUF - 1) % NBUF
            jn = j + NBUF - 1

            @pl.when(jn < CHUNKS)
            def _():
                istart(jn, bn)

            iwait(j, b)
            pltpu.sync_copy(ones_v, acc.at[idx[b]], add=True)

        return 0

    lax.fori_loop(0, CHUNKS // NBUF, chunk, 0)
    plsc.subcore_barrier()

    r = pl.multiple_of(s * RPT, 8)
    pltpu.sync_copy(acc.at[pl.ds(r, RPT)], out_hbm.at[c, pl.ds(r, RPT)])


_sc_deg = pl.kernel(
    _sc_deg_body,
    out_type=jax.ShapeDtypeStruct((NC, Np, DW), f32),
    mesh=_mesh,
    scratch_types=(
        [pltpu.VMEM((CH, DW), f32),
         pltpu.VMEM((CH, DW), f32)]
        + [pltpu.VMEM((CH,), jnp.int32) for _ in range(NBUF)]
        + [pltpu.VMEM_SHARED((Np, DW), f32)]
        + [pltpu.SemaphoreType.DMA for _ in range(NBUF)]
    ),
)


def _sc_prop_body(g_hbm, src_hbm, dst_hbm, out_hbm,
                  sidx0, sidx1, sidx2, sidx3,
                  didx0, didx1, didx2, didx3,
                  rows0, rows1, acc,
                  isem0, isem1, isem2, isem3,
                  gsem0, gsem1):
    c = lax.axis_index("c")
    s = lax.axis_index("s")
    wid = c * NS + s
    sidx = (sidx0, sidx1, sidx2, sidx3)
    didx = (didx0, didx1, didx2, didx3)
    rows = (rows0, rows1)
    isem = (isem0, isem1, isem2, isem3)
    gsem = (gsem0, gsem1)

    zrow = jnp.zeros((16,), f32)

    def fill(i, _):
        for q in range(H // 16):
            rows0[i, pl.ds(16 * q, 16)] = zrow
        return 0

    lax.fori_loop(0, CH, fill, 0)

    def zloop(k, _):
        r = pl.multiple_of(s * RPT + k * CH, 8)
        pltpu.sync_copy(rows0, acc.at[pl.ds(r, CH)])
        return 0

    lax.fori_loop(0, RPT // CH, zloop, 0)
    plsc.subcore_barrier()

    def istart(j, b):
        base = pl.multiple_of(wid * EPT + j * CH, 8)
        pltpu.async_copy(src_hbm.at[pl.ds(base, CH)], sidx[b], isem[b])
        pltpu.async_copy(dst_hbm.at[pl.ds(base, CH)], didx[b], isem[b])

    def iwait(j, b):
        base = pl.multiple_of(wid * EPT + j * CH, 8)
        pltpu.make_async_copy(src_hbm.at[pl.ds(base, CH)], sidx[b],
                              isem[b]).wait()
        pltpu.make_async_copy(dst_hbm.at[pl.ds(base, CH)], didx[b],
                              isem[b]).wait()

    def gstart(ib, rb):
        pltpu.async_copy(g_hbm.at[sidx[ib]], rows[rb], gsem[rb])

    def gwait(ib, rb):
        pltpu.make_async_copy(g_hbm.at[sidx[ib]], rows[rb], gsem[rb]).wait()

    for b in range(NBUF - 1):
        istart(b, b)
    iwait(0, 0)
    gstart(0, 0)

    def body(t, _):
        for u in range(NBUF):
            j = NBUF * t + u
            b = u
            rb = u % 2
            rb1 = (u + 1) % 2
            b1 = (u + 1) % NBUF
            b3 = (u + NBUF - 1) % NBUF
            jn = j + NBUF - 1

            @pl.when(jn < CHUNKS)
            def _():
                istart(jn, b3)

            @pl.when(j + 1 < CHUNKS)
            def _():
                iwait(j + 1, b1)
                gstart(b1, rb1)

            gwait(b, rb)
            pltpu.sync_copy(rows[rb], acc.at[didx[b]], add=True)

        return 0

    lax.fori_loop(0, CHUNKS // NBUF, body, 0)
    plsc.subcore_barrier()

    r = pl.multiple_of(s * RPT, 8)
    pltpu.sync_copy(acc.at[pl.ds(r, RPT)], out_hbm.at[c, pl.ds(r, RPT)])


_sc_prop = pl.kernel(
    _sc_prop_body,
    out_type=jax.ShapeDtypeStruct((NC, Np, H), f32),
    mesh=_mesh,
    scratch_types=(
        [pltpu.VMEM((CH,), jnp.int32) for _ in range(2 * NBUF)]
        + [pltpu.VMEM((CH, H), f32) for _ in range(2)]
        + [pltpu.VMEM_SHARED((Np, H), f32)]
        + [pltpu.SemaphoreType.DMA for _ in range(NBUF + 2)]
    ),
)


BR = 2048
_PREC = lax.Precision.HIGHEST


def _tca_body(x_ref, Win_ref, bin_ref, h_ref):
    h_ref[...] = jnp.dot(x_ref[...], Win_ref[...], precision=_PREC,
                         preferred_element_type=f32) + bin_ref[...]


_tca = pl.pallas_call(
    _tca_body,
    grid=(Np // BR,),
    in_specs=[
        pl.BlockSpec((BR, D_IN), lambda i: (i, 0)),
        pl.BlockSpec((D_IN, H), lambda i: (0, 0)),
        pl.BlockSpec((1, H), lambda i: (0, 0)),
    ],
    out_specs=pl.BlockSpec((BR, H), lambda i: (i, 0)),
    out_shape=jax.ShapeDtypeStruct((Np, H), f32),
)


def _tcb_body(sdeg_ref, h_ref, W0_ref, dinv_ref, g_ref):
    deg = 1.0 + sdeg_ref[0, :, 0:1] + sdeg_ref[1, :, 0:1]
    dinv = lax.rsqrt(deg)
    dinv_ref[...] = dinv
    g_ref[...] = dinv * jnp.dot(h_ref[...], W0_ref[...], precision=_PREC,
                                preferred_element_type=f32)


_tcb = pl.pallas_call(
    _tcb_body,
    grid=(Np // BR,),
    in_specs=[
        pl.BlockSpec((NC, BR, DW), lambda i: (0, i, 0)),
        pl.BlockSpec((BR, H), lambda i: (i, 0)),
        pl.BlockSpec((H, H), lambda i: (0, 0)),
    ],
    out_specs=[
        pl.BlockSpec((BR, 1), lambda i: (i, 0)),
        pl.BlockSpec((BR, H), lambda i: (i, 0)),
    ],
    out_shape=[
        jax.ShapeDtypeStruct((Np, 1), f32),
        jax.ShapeDtypeStruct((Np, H), f32),
    ],
)


def _tcmid_body(s_ref, g_ref, h_ref, dinv_ref, cb_ref, ab_ref, Wn_ref,
                ho_ref, go_ref):
    dinv = dinv_ref[...]
    u = jnp.tanh(dinv * (s_ref[0] + s_ref[1] + g_ref[...]) + cb_ref[...])
    hn = ab_ref[0, 1] * h_ref[...] + ab_ref[0, 0] * u
    ho_ref[...] = hn
    go_ref[...] = dinv * jnp.dot(hn, Wn_ref[...], precision=_PREC,
                                 preferred_element_type=f32)


_tcmid = pl.pallas_call(
    _tcmid_body,
    grid=(Np // BR,),
    in_specs=[
        pl.BlockSpec((NC, BR, H), lambda i: (0, i, 0)),
        pl.BlockSpec((BR, H), lambda i: (i, 0)),
        pl.BlockSpec((BR, H), lambda i: (i, 0)),
        pl.BlockSpec((BR, 1), lambda i: (i, 0)),
        pl.BlockSpec((1, H), lambda i: (0, 0)),
        pl.BlockSpec((1, 2), lambda i: (0, 0)),
        pl.BlockSpec((H, H), lambda i: (0, 0)),
    ],
    out_specs=[
        pl.BlockSpec((BR, H), lambda i: (i, 0)),
        pl.BlockSpec((BR, H), lambda i: (i, 0)),
    ],
    out_shape=[
        jax.ShapeDtypeStruct((Np, H), f32),
        jax.ShapeDtypeStruct((Np, H), f32),
    ],
)


def _tcfin_body(s_ref, g_ref, h_ref, dinv_ref, cb_ref, ab_ref, Wout_ref,
                bout_ref, o_ref):
    dinv = dinv_ref[...]
    u = jnp.tanh(dinv * (s_ref[0] + s_ref[1] + g_ref[...]) + cb_ref[...])
    hn = ab_ref[0, 1] * h_ref[...] + ab_ref[0, 0] * u
    logits = jnp.dot(hn, Wout_ref[...], precision=_PREC,
                     preferred_element_type=f32) + bout_ref[...]
    m = jnp.max(logits, axis=1, keepdims=True)
    lse = m + jnp.log(jnp.sum(jnp.exp(logits - m), axis=1, keepdims=True))
    o_ref[...] = logits - lse


_tcfin = pl.pallas_call(
    _tcfin_body,
    grid=(Np // BR,),
    in_specs=[
        pl.BlockSpec((NC, BR, H), lambda i: (0, i, 0)),
        pl.BlockSpec((BR, H), lambda i: (i, 0)),
        pl.BlockSpec((BR, H), lambda i: (i, 0)),
        pl.BlockSpec((BR, 1), lambda i: (i, 0)),
        pl.BlockSpec((1, H), lambda i: (0, 0)),
        pl.BlockSpec((1, 2), lambda i: (0, 0)),
        pl.BlockSpec((H, OUT), lambda i: (0, 0)),
        pl.BlockSpec((1, OUT), lambda i: (0, 0)),
    ],
    out_specs=pl.BlockSpec((BR, OUT), lambda i: (i, 0)),
    out_shape=jax.ShapeDtypeStruct((Np, OUT), f32),
)


@jax.jit
def kernel(x, edge_index, W_in, b_in, convW, convb, alpha, beta, W_out,
           b_out):
    xp = jnp.pad(x, ((0, Np - N), (0, 0)))
    pad_idx = jnp.full((Ep - E,), N, jnp.int32)
    srcp = jnp.concatenate([edge_index[0], pad_idx])
    dstp = jnp.concatenate([edge_index[1], pad_idx])

    sdeg = _sc_deg(dstp)
    h = _tca(xp, W_in, b_in.reshape(1, H))
    dinv, g = _tcb(sdeg, h, convW[0])

    ab = jnp.stack([alpha, beta], axis=1)
    for i in range(L):
        sacc = _sc_prop(g, srcp, dstp)
        cb = convb[i].reshape(1, H)
        abi = ab[i].reshape(1, 2)
        if i < L - 1:
            h, g = _tcmid(sacc, g, h, dinv, cb, abi, convW[i + 1])
        else:
            o = _tcfin(sacc, g, h, dinv, cb, abi, W_out,
                       b_out.reshape(1, OUT))
    return o[:N]
